# baseline (speedup 1.0000x reference)
import functools

import jax
import jax.numpy as jnp
from jax.experimental import pallas as pl
from jax.experimental.pallas import tpu as pltpu

QB = 64
KBLK = 4096
LANES = 128
NLVL = 4
TOPK = 5


def _body(bs_ref, q_ref, k_ref, vals_ref, idx_ref, accv_ref, acci_ref,
          *, n_k, nkb):
    kb = pl.program_id(1)

    @pl.when(kb == 0)
    def _init():
        accv_ref[...] = jnp.full(accv_ref.shape, -jnp.inf, jnp.float32)
        acci_ref[...] = jnp.zeros(acci_ref.shape, jnp.int32)

    q = q_ref[...]
    qn = q / jnp.sqrt(jnp.sum(q * q, axis=1, keepdims=True))
    k = k_ref[0]
    kn = k / jnp.sqrt(jnp.sum(k * k, axis=1, keepdims=True))
    scores = jax.lax.dot_general(
        qn, kn, (((1,), (1,)), ((), ())),
        preferred_element_type=jnp.float32)
    gidx = kb * KBLK + jax.lax.broadcasted_iota(jnp.int32, (QB, KBLK), 1)
    scores = jnp.where(gidx < n_k, scores, -jnp.inf)

    for c in range(KBLK // LANES):
        v = scores[:, c * LANES:(c + 1) * LANES]
        vi = gidx[:, c * LANES:(c + 1) * LANES]
        for j in range(NLVL):
            av = accv_ref[j]
            ai = acci_ref[j]
            gt = v > av
            accv_ref[j] = jnp.where(gt, v, av)
            acci_ref[j] = jnp.where(gt, vi, ai)
            v = jnp.where(gt, av, v)
            vi = jnp.where(gt, ai, vi)

    @pl.when(kb == nkb - 1)
    def _extract():
        Vw = [accv_ref[j] for j in range(NLVL)]
        Iw = [acci_ref[j] for j in range(NLVL)]
        outv, outi = [], []
        for _r in range(TOPK):
            M, MI = Vw[0], Iw[0]
            for j in range(1, NLVL):
                better = (Vw[j] > M) | ((Vw[j] == M) & (Iw[j] < MI))
                M = jnp.where(better, Vw[j], M)
                MI = jnp.where(better, Iw[j], MI)
            m = jnp.max(M, axis=1, keepdims=True)
            mi = jnp.min(jnp.where(M == m, MI, jnp.int32(2**31 - 1)),
                         axis=1, keepdims=True)
            outv.append(m)
            outi.append(mi)
            for j in range(NLVL):
                hit = (Vw[j] == m) & (Iw[j] == mi)
                Vw[j] = jnp.where(hit, -jnp.inf, Vw[j])
        vals_ref[...] = jnp.concatenate(outv, axis=1)
        idx_ref[...] = jnp.concatenate(outi, axis=1)


def kernel(batch_inputs, batch_query, batch_style, keys, topk):
    del batch_inputs, topk
    q_n, d = batch_query.shape
    s_n, k_n, _ = keys.shape
    nb = q_n // QB + 1
    nkb = (k_n + KBLK - 1) // KBLK

    style = batch_style.astype(jnp.int32)
    is0 = (style == 0).astype(jnp.int32)
    c0 = jnp.cumsum(is0)
    c1 = jnp.cumsum(1 - is0)
    n0 = c0[-1]
    ceil0 = (n0 + QB - 1) // QB
    padpos = jnp.where(style == 0, c0 - 1, ceil0 * QB + c1 - 1)
    perm = jnp.zeros((nb * QB,), jnp.int32).at[padpos].set(
        jnp.arange(q_n, dtype=jnp.int32), mode="drop")
    qs = batch_query[perm]
    bstyle = (jnp.arange(nb, dtype=jnp.int32) >= ceil0).astype(jnp.int32)

    body = functools.partial(_body, n_k=k_n, nkb=nkb)
    grid_spec = pltpu.PrefetchScalarGridSpec(
        num_scalar_prefetch=1,
        grid=(nb, nkb),
        in_specs=[
            pl.BlockSpec((QB, d), lambda b, kb, bs: (b, 0)),
            pl.BlockSpec((1, KBLK, d), lambda b, kb, bs: (bs[b], kb, 0)),
        ],
        out_specs=(
            pl.BlockSpec((QB, TOPK), lambda b, kb, bs: (b, 0)),
            pl.BlockSpec((QB, TOPK), lambda b, kb, bs: (b, 0)),
        ),
        scratch_shapes=[
            pltpu.VMEM((NLVL, QB, LANES), jnp.float32),
            pltpu.VMEM((NLVL, QB, LANES), jnp.int32),
        ],
    )
    vals_p, idx_p = pl.pallas_call(
        body,
        grid_spec=grid_spec,
        out_shape=(
            jax.ShapeDtypeStruct((nb * QB, TOPK), jnp.float32),
            jax.ShapeDtypeStruct((nb * QB, TOPK), jnp.int32),
        ),
        compiler_params=pltpu.CompilerParams(
            dimension_semantics=("parallel", "arbitrary")),
        interpret=False,
    )(bstyle, qs, keys)

    return vals_p[padpos], idx_p[padpos]


# speedup vs baseline: 1.3546x; 1.3546x over previous
import functools

import jax
import jax.numpy as jnp
from jax.experimental import pallas as pl
from jax.experimental.pallas import tpu as pltpu

QB = 256
KBLK = 4096
LANES = 128
NLVL = 4
TOPK = 5


def _body(bs_ref, q_ref, k_ref, vals_ref, idx_ref, accv_ref, acci_ref,
          *, n_k, nkb):
    kb = pl.program_id(1)

    @pl.when(kb == 0)
    def _init():
        accv_ref[...] = jnp.full(accv_ref.shape, -jnp.inf, jnp.float32)
        acci_ref[...] = jnp.zeros(acci_ref.shape, jnp.int32)

    q = q_ref[...]
    qn = q / jnp.sqrt(jnp.sum(q * q, axis=1, keepdims=True))
    k = k_ref[0]
    kn = k / jnp.sqrt(jnp.sum(k * k, axis=1, keepdims=True))
    scores = jax.lax.dot_general(
        qn, kn, (((1,), (1,)), ((), ())),
        preferred_element_type=jnp.float32)
    gidx = kb * KBLK + jax.lax.broadcasted_iota(jnp.int32, (QB, KBLK), 1)
    scores = jnp.where(gidx < n_k, scores, -jnp.inf)

    for c in range(KBLK // LANES):
        v = scores[:, c * LANES:(c + 1) * LANES]
        vi = gidx[:, c * LANES:(c + 1) * LANES]
        for j in range(NLVL):
            av = accv_ref[j]
            ai = acci_ref[j]
            gt = v > av
            accv_ref[j] = jnp.where(gt, v, av)
            acci_ref[j] = jnp.where(gt, vi, ai)
            v = jnp.where(gt, av, v)
            vi = jnp.where(gt, ai, vi)

    @pl.when(kb == nkb - 1)
    def _extract():
        Vw = [accv_ref[j] for j in range(NLVL)]
        Iw = [acci_ref[j] for j in range(NLVL)]
        outv, outi = [], []
        for _r in range(TOPK):
            M, MI = Vw[0], Iw[0]
            for j in range(1, NLVL):
                better = (Vw[j] > M) | ((Vw[j] == M) & (Iw[j] < MI))
                M = jnp.where(better, Vw[j], M)
                MI = jnp.where(better, Iw[j], MI)
            m = jnp.max(M, axis=1, keepdims=True)
            mi = jnp.min(jnp.where(M == m, MI, jnp.int32(2**31 - 1)),
                         axis=1, keepdims=True)
            outv.append(m)
            outi.append(mi)
            for j in range(NLVL):
                hit = (Vw[j] == m) & (Iw[j] == mi)
                Vw[j] = jnp.where(hit, -jnp.inf, Vw[j])
        vals_ref[...] = jnp.concatenate(outv, axis=1)
        idx_ref[...] = jnp.concatenate(outi, axis=1)


def kernel(batch_inputs, batch_query, batch_style, keys, topk):
    del batch_inputs, topk
    q_n, d = batch_query.shape
    s_n, k_n, _ = keys.shape
    nb = q_n // QB + 1
    nkb = (k_n + KBLK - 1) // KBLK

    style = batch_style.astype(jnp.int32)
    is0 = (style == 0).astype(jnp.int32)
    c0 = jnp.cumsum(is0)
    c1 = jnp.cumsum(1 - is0)
    n0 = c0[-1]
    ceil0 = (n0 + QB - 1) // QB
    padpos = jnp.where(style == 0, c0 - 1, ceil0 * QB + c1 - 1)
    perm = jnp.zeros((nb * QB,), jnp.int32).at[padpos].set(
        jnp.arange(q_n, dtype=jnp.int32), mode="drop")
    qs = batch_query[perm]
    bstyle = (jnp.arange(nb, dtype=jnp.int32) >= ceil0).astype(jnp.int32)

    body = functools.partial(_body, n_k=k_n, nkb=nkb)
    grid_spec = pltpu.PrefetchScalarGridSpec(
        num_scalar_prefetch=1,
        grid=(nb, nkb),
        in_specs=[
            pl.BlockSpec((QB, d), lambda b, kb, bs: (b, 0)),
            pl.BlockSpec((1, KBLK, d), lambda b, kb, bs: (bs[b], kb, 0)),
        ],
        out_specs=(
            pl.BlockSpec((QB, TOPK), lambda b, kb, bs: (b, 0)),
            pl.BlockSpec((QB, TOPK), lambda b, kb, bs: (b, 0)),
        ),
        scratch_shapes=[
            pltpu.VMEM((NLVL, QB, LANES), jnp.float32),
            pltpu.VMEM((NLVL, QB, LANES), jnp.int32),
        ],
    )
    vals_p, idx_p = pl.pallas_call(
        body,
        grid_spec=grid_spec,
        out_shape=(
            jax.ShapeDtypeStruct((nb * QB, TOPK), jnp.float32),
            jax.ShapeDtypeStruct((nb * QB, TOPK), jnp.int32),
        ),
        compiler_params=pltpu.CompilerParams(
            dimension_semantics=("parallel", "arbitrary")),
        interpret=False,
    )(bstyle, qs, keys)

    return vals_p[padpos], idx_p[padpos]


# speedup vs baseline: 1.4118x; 1.0422x over previous
import functools

import jax
import jax.numpy as jnp
from jax.experimental import pallas as pl
from jax.experimental.pallas import tpu as pltpu

QB = 256
KBLK = 6400
LANES = 128
NLVL = 4
TOPK = 5


def _body(bs_ref, q_ref, k_ref, vals_ref, idx_ref, accv_ref, acci_ref,
          *, n_k, nkb):
    kb = pl.program_id(1)

    @pl.when(kb == 0)
    def _init():
        accv_ref[...] = jnp.full(accv_ref.shape, -jnp.inf, jnp.float32)
        acci_ref[...] = jnp.zeros(acci_ref.shape, jnp.int32)

    q = q_ref[...]
    qn = q / jnp.sqrt(jnp.sum(q * q, axis=1, keepdims=True))
    k = k_ref[0]
    kn = k / jnp.sqrt(jnp.sum(k * k, axis=1, keepdims=True))
    scores = jax.lax.dot_general(
        qn, kn, (((1,), (1,)), ((), ())),
        preferred_element_type=jnp.float32)
    gidx = kb * KBLK + jax.lax.broadcasted_iota(jnp.int32, (QB, KBLK), 1)
    scores = jnp.where(gidx < n_k, scores, -jnp.inf)

    for c in range(KBLK // LANES):
        v = scores[:, c * LANES:(c + 1) * LANES]
        vi = gidx[:, c * LANES:(c + 1) * LANES]
        for j in range(NLVL):
            av = accv_ref[j]
            ai = acci_ref[j]
            gt = v > av
            accv_ref[j] = jnp.where(gt, v, av)
            acci_ref[j] = jnp.where(gt, vi, ai)
            v = jnp.where(gt, av, v)
            vi = jnp.where(gt, ai, vi)

    @pl.when(kb == nkb - 1)
    def _extract():
        Vw = [accv_ref[j] for j in range(NLVL)]
        Iw = [acci_ref[j] for j in range(NLVL)]
        outv, outi = [], []
        for _r in range(TOPK):
            M, MI = Vw[0], Iw[0]
            for j in range(1, NLVL):
                better = (Vw[j] > M) | ((Vw[j] == M) & (Iw[j] < MI))
                M = jnp.where(better, Vw[j], M)
                MI = jnp.where(better, Iw[j], MI)
            m = jnp.max(M, axis=1, keepdims=True)
            mi = jnp.min(jnp.where(M == m, MI, jnp.int32(2**31 - 1)),
                         axis=1, keepdims=True)
            outv.append(m)
            outi.append(mi)
            for j in range(NLVL):
                hit = (Vw[j] == m) & (Iw[j] == mi)
                Vw[j] = jnp.where(hit, -jnp.inf, Vw[j])
        vals_ref[...] = jnp.concatenate(outv, axis=1)
        idx_ref[...] = jnp.concatenate(outi, axis=1)


def kernel(batch_inputs, batch_query, batch_style, keys, topk):
    del batch_inputs, topk
    q_n, d = batch_query.shape
    s_n, k_n, _ = keys.shape
    nb = q_n // QB + 1
    nkb = (k_n + KBLK - 1) // KBLK

    style = batch_style.astype(jnp.int32)
    is0 = (style == 0).astype(jnp.int32)
    c0 = jnp.cumsum(is0)
    c1 = jnp.cumsum(1 - is0)
    n0 = c0[-1]
    ceil0 = (n0 + QB - 1) // QB
    padpos = jnp.where(style == 0, c0 - 1, ceil0 * QB + c1 - 1)
    perm = jnp.zeros((nb * QB,), jnp.int32).at[padpos].set(
        jnp.arange(q_n, dtype=jnp.int32), mode="drop")
    qs = batch_query[perm]
    bstyle = (jnp.arange(nb, dtype=jnp.int32) >= ceil0).astype(jnp.int32)

    body = functools.partial(_body, n_k=k_n, nkb=nkb)
    grid_spec = pltpu.PrefetchScalarGridSpec(
        num_scalar_prefetch=1,
        grid=(nb, nkb),
        in_specs=[
            pl.BlockSpec((QB, d), lambda b, kb, bs: (b, 0)),
            pl.BlockSpec((1, KBLK, d), lambda b, kb, bs: (bs[b], kb, 0)),
        ],
        out_specs=(
            pl.BlockSpec((QB, TOPK), lambda b, kb, bs: (b, 0)),
            pl.BlockSpec((QB, TOPK), lambda b, kb, bs: (b, 0)),
        ),
        scratch_shapes=[
            pltpu.VMEM((NLVL, QB, LANES), jnp.float32),
            pltpu.VMEM((NLVL, QB, LANES), jnp.int32),
        ],
    )
    vals_p, idx_p = pl.pallas_call(
        body,
        grid_spec=grid_spec,
        out_shape=(
            jax.ShapeDtypeStruct((nb * QB, TOPK), jnp.float32),
            jax.ShapeDtypeStruct((nb * QB, TOPK), jnp.int32),
        ),
        compiler_params=pltpu.CompilerParams(
            dimension_semantics=("parallel", "arbitrary")),
        interpret=False,
    )(bstyle, qs, keys)

    return vals_p[padpos], idx_p[padpos]


# speedup vs baseline: 1.4398x; 1.0199x over previous
import functools

import jax
import jax.numpy as jnp
from jax.experimental import pallas as pl
from jax.experimental.pallas import tpu as pltpu

QB = 256
KBLK = 12800
LANES = 128
NLVL = 4
TOPK = 5


def _body(bs_ref, q_ref, k_ref, vals_ref, idx_ref, accv_ref, acci_ref,
          *, n_k, nkb):
    kb = pl.program_id(1)

    @pl.when(kb == 0)
    def _init():
        accv_ref[...] = jnp.full(accv_ref.shape, -jnp.inf, jnp.float32)
        acci_ref[...] = jnp.zeros(acci_ref.shape, jnp.int32)

    q = q_ref[...]
    qn = q / jnp.sqrt(jnp.sum(q * q, axis=1, keepdims=True))
    k = k_ref[0]
    kn = k / jnp.sqrt(jnp.sum(k * k, axis=1, keepdims=True))
    scores = jax.lax.dot_general(
        qn, kn, (((1,), (1,)), ((), ())),
        preferred_element_type=jnp.float32)
    gidx = kb * KBLK + jax.lax.broadcasted_iota(jnp.int32, (QB, KBLK), 1)
    scores = jnp.where(gidx < n_k, scores, -jnp.inf)

    for c in range(KBLK // LANES):
        v = scores[:, c * LANES:(c + 1) * LANES]
        vi = gidx[:, c * LANES:(c + 1) * LANES]
        for j in range(NLVL):
            av = accv_ref[j]
            ai = acci_ref[j]
            gt = v > av
            accv_ref[j] = jnp.where(gt, v, av)
            acci_ref[j] = jnp.where(gt, vi, ai)
            v = jnp.where(gt, av, v)
            vi = jnp.where(gt, ai, vi)

    @pl.when(kb == nkb - 1)
    def _extract():
        Vw = [accv_ref[j] for j in range(NLVL)]
        Iw = [acci_ref[j] for j in range(NLVL)]
        outv, outi = [], []
        for _r in range(TOPK):
            M, MI = Vw[0], Iw[0]
            for j in range(1, NLVL):
                better = (Vw[j] > M) | ((Vw[j] == M) & (Iw[j] < MI))
                M = jnp.where(better, Vw[j], M)
                MI = jnp.where(better, Iw[j], MI)
            m = jnp.max(M, axis=1, keepdims=True)
            mi = jnp.min(jnp.where(M == m, MI, jnp.int32(2**31 - 1)),
                         axis=1, keepdims=True)
            outv.append(m)
            outi.append(mi)
            for j in range(NLVL):
                hit = (Vw[j] == m) & (Iw[j] == mi)
                Vw[j] = jnp.where(hit, -jnp.inf, Vw[j])
        vals_ref[...] = jnp.concatenate(outv, axis=1)
        idx_ref[...] = jnp.concatenate(outi, axis=1)


def kernel(batch_inputs, batch_query, batch_style, keys, topk):
    del batch_inputs, topk
    q_n, d = batch_query.shape
    s_n, k_n, _ = keys.shape
    nb = q_n // QB + 1
    nkb = (k_n + KBLK - 1) // KBLK

    style = batch_style.astype(jnp.int32)
    is0 = (style == 0).astype(jnp.int32)
    c0 = jnp.cumsum(is0)
    c1 = jnp.cumsum(1 - is0)
    n0 = c0[-1]
    ceil0 = (n0 + QB - 1) // QB
    padpos = jnp.where(style == 0, c0 - 1, ceil0 * QB + c1 - 1)
    perm = jnp.zeros((nb * QB,), jnp.int32).at[padpos].set(
        jnp.arange(q_n, dtype=jnp.int32), mode="drop")
    qs = batch_query[perm]
    bstyle = (jnp.arange(nb, dtype=jnp.int32) >= ceil0).astype(jnp.int32)

    body = functools.partial(_body, n_k=k_n, nkb=nkb)
    grid_spec = pltpu.PrefetchScalarGridSpec(
        num_scalar_prefetch=1,
        grid=(nb, nkb),
        in_specs=[
            pl.BlockSpec((QB, d), lambda b, kb, bs: (b, 0)),
            pl.BlockSpec((1, KBLK, d), lambda b, kb, bs: (bs[b], kb, 0)),
        ],
        out_specs=(
            pl.BlockSpec((QB, TOPK), lambda b, kb, bs: (b, 0)),
            pl.BlockSpec((QB, TOPK), lambda b, kb, bs: (b, 0)),
        ),
        scratch_shapes=[
            pltpu.VMEM((NLVL, QB, LANES), jnp.float32),
            pltpu.VMEM((NLVL, QB, LANES), jnp.int32),
        ],
    )
    vals_p, idx_p = pl.pallas_call(
        body,
        grid_spec=grid_spec,
        out_shape=(
            jax.ShapeDtypeStruct((nb * QB, TOPK), jnp.float32),
            jax.ShapeDtypeStruct((nb * QB, TOPK), jnp.int32),
        ),
        compiler_params=pltpu.CompilerParams(
            dimension_semantics=("parallel", "arbitrary")),
        interpret=False,
    )(bstyle, qs, keys)

    return vals_p[padpos], idx_p[padpos]


# speedup vs baseline: 1.4432x; 1.0023x over previous
import functools

import jax
import jax.numpy as jnp
from jax.experimental import pallas as pl
from jax.experimental.pallas import tpu as pltpu

QB = 256
KBLK = 25600
LANES = 128
NLVL = 4
TOPK = 5


def _body(bs_ref, q_ref, k_ref, vals_ref, idx_ref, accv_ref, acci_ref,
          *, n_k, nkb):
    kb = pl.program_id(1)

    @pl.when(kb == 0)
    def _init():
        accv_ref[...] = jnp.full(accv_ref.shape, -jnp.inf, jnp.float32)
        acci_ref[...] = jnp.zeros(acci_ref.shape, jnp.int32)

    q = q_ref[...]
    qn = q / jnp.sqrt(jnp.sum(q * q, axis=1, keepdims=True))
    k = k_ref[0]
    kn = k / jnp.sqrt(jnp.sum(k * k, axis=1, keepdims=True))
    scores = jax.lax.dot_general(
        qn, kn, (((1,), (1,)), ((), ())),
        preferred_element_type=jnp.float32)
    gidx = kb * KBLK + jax.lax.broadcasted_iota(jnp.int32, (QB, KBLK), 1)
    scores = jnp.where(gidx < n_k, scores, -jnp.inf)

    for c in range(KBLK // LANES):
        v = scores[:, c * LANES:(c + 1) * LANES]
        vi = gidx[:, c * LANES:(c + 1) * LANES]
        for j in range(NLVL):
            av = accv_ref[j]
            ai = acci_ref[j]
            gt = v > av
            accv_ref[j] = jnp.where(gt, v, av)
            acci_ref[j] = jnp.where(gt, vi, ai)
            if j < NLVL - 1:
                v = jnp.where(gt, av, v)
                vi = jnp.where(gt, ai, vi)

    @pl.when(kb == nkb - 1)
    def _extract():
        Vw = [accv_ref[j] for j in range(NLVL)]
        Iw = [acci_ref[j] for j in range(NLVL)]
        outv, outi = [], []
        for _r in range(TOPK):
            M, MI = Vw[0], Iw[0]
            for j in range(1, NLVL):
                better = (Vw[j] > M) | ((Vw[j] == M) & (Iw[j] < MI))
                M = jnp.where(better, Vw[j], M)
                MI = jnp.where(better, Iw[j], MI)
            m = jnp.max(M, axis=1, keepdims=True)
            mi = jnp.min(jnp.where(M == m, MI, jnp.int32(2**31 - 1)),
                         axis=1, keepdims=True)
            outv.append(m)
            outi.append(mi)
            for j in range(NLVL):
                hit = (Vw[j] == m) & (Iw[j] == mi)
                Vw[j] = jnp.where(hit, -jnp.inf, Vw[j])
        vals_ref[...] = jnp.concatenate(outv, axis=1)
        idx_ref[...] = jnp.concatenate(outi, axis=1)


def kernel(batch_inputs, batch_query, batch_style, keys, topk):
    del batch_inputs, topk
    q_n, d = batch_query.shape
    s_n, k_n, _ = keys.shape
    nb = q_n // QB + 1
    nkb = (k_n + KBLK - 1) // KBLK

    style = batch_style.astype(jnp.int32)
    is0 = (style == 0).astype(jnp.int32)
    c0 = jnp.cumsum(is0)
    c1 = jnp.cumsum(1 - is0)
    n0 = c0[-1]
    ceil0 = (n0 + QB - 1) // QB
    padpos = jnp.where(style == 0, c0 - 1, ceil0 * QB + c1 - 1)
    perm = jnp.zeros((nb * QB,), jnp.int32).at[padpos].set(
        jnp.arange(q_n, dtype=jnp.int32), mode="drop")
    qs = batch_query[perm]
    bstyle = (jnp.arange(nb, dtype=jnp.int32) >= ceil0).astype(jnp.int32)

    body = functools.partial(_body, n_k=k_n, nkb=nkb)
    grid_spec = pltpu.PrefetchScalarGridSpec(
        num_scalar_prefetch=1,
        grid=(nb, nkb),
        in_specs=[
            pl.BlockSpec((QB, d), lambda b, kb, bs: (b, 0)),
            pl.BlockSpec((1, KBLK, d), lambda b, kb, bs: (bs[b], kb, 0)),
        ],
        out_specs=(
            pl.BlockSpec((QB, TOPK), lambda b, kb, bs: (b, 0)),
            pl.BlockSpec((QB, TOPK), lambda b, kb, bs: (b, 0)),
        ),
        scratch_shapes=[
            pltpu.VMEM((NLVL, QB, LANES), jnp.float32),
            pltpu.VMEM((NLVL, QB, LANES), jnp.int32),
        ],
    )
    vals_p, idx_p = pl.pallas_call(
        body,
        grid_spec=grid_spec,
        out_shape=(
            jax.ShapeDtypeStruct((nb * QB, TOPK), jnp.float32),
            jax.ShapeDtypeStruct((nb * QB, TOPK), jnp.int32),
        ),
        compiler_params=pltpu.CompilerParams(
            dimension_semantics=("parallel", "arbitrary")),
        interpret=False,
    )(bstyle, qs, keys)

    return vals_p[padpos], idx_p[padpos]


# speedup vs baseline: 1.4781x; 1.0242x over previous
import functools

import jax
import jax.numpy as jnp
from jax.experimental import pallas as pl
from jax.experimental.pallas import tpu as pltpu

QB = 256
KBLK = 25600
LANES = 128
NLVL = 4
TOPK = 5


def _body(bs_ref, q_ref, k_ref, vals_ref, idx_ref, accv_ref, acci_ref,
          *, n_k, nkb):
    kb = pl.program_id(1)

    @pl.when(kb == 0)
    def _init():
        accv_ref[...] = jnp.full(accv_ref.shape, -jnp.inf, jnp.float32)
        acci_ref[...] = jnp.zeros(acci_ref.shape, jnp.int32)

    q = q_ref[...]
    qn = q / jnp.sqrt(jnp.sum(q * q, axis=1, keepdims=True))
    k = k_ref[0]
    kn = k / jnp.sqrt(jnp.sum(k * k, axis=1, keepdims=True))
    scores = jax.lax.dot_general(
        qn, kn, (((1,), (1,)), ((), ())),
        preferred_element_type=jnp.float32)
    gidx = kb * KBLK + jax.lax.broadcasted_iota(jnp.int32, (QB, KBLK), 1)
    safe_c = max((n_k - (nkb - 1) * KBLK) // LANES, 0)

    for c in range(KBLK // LANES):
        v = scores[:, c * LANES:(c + 1) * LANES]
        vi = gidx[:, c * LANES:(c + 1) * LANES]
        if c >= safe_c:
            v = jnp.where(vi < n_k, v, -jnp.inf)
        for j in range(NLVL):
            av = accv_ref[j]
            ai = acci_ref[j]
            gt = v > av
            accv_ref[j] = jnp.where(gt, v, av)
            acci_ref[j] = jnp.where(gt, vi, ai)
            if j < NLVL - 1:
                v = jnp.where(gt, av, v)
                vi = jnp.where(gt, ai, vi)

    @pl.when(kb == nkb - 1)
    def _extract():
        Vw = [accv_ref[j] for j in range(NLVL)]
        Iw = [acci_ref[j] for j in range(NLVL)]
        outv, outi = [], []
        for _r in range(TOPK):
            M, MI = Vw[0], Iw[0]
            for j in range(1, NLVL):
                better = (Vw[j] > M) | ((Vw[j] == M) & (Iw[j] < MI))
                M = jnp.where(better, Vw[j], M)
                MI = jnp.where(better, Iw[j], MI)
            m = jnp.max(M, axis=1, keepdims=True)
            mi = jnp.min(jnp.where(M == m, MI, jnp.int32(2**31 - 1)),
                         axis=1, keepdims=True)
            outv.append(m)
            outi.append(mi)
            for j in range(NLVL):
                hit = (Vw[j] == m) & (Iw[j] == mi)
                Vw[j] = jnp.where(hit, -jnp.inf, Vw[j])
        vals_ref[...] = jnp.concatenate(outv, axis=1)
        idx_ref[...] = jnp.concatenate(outi, axis=1)


def kernel(batch_inputs, batch_query, batch_style, keys, topk):
    del batch_inputs, topk
    q_n, d = batch_query.shape
    s_n, k_n, _ = keys.shape
    nb = q_n // QB + 1
    nkb = (k_n + KBLK - 1) // KBLK

    style = batch_style.astype(jnp.int32)
    is0 = (style == 0).astype(jnp.int32)
    c0 = jnp.cumsum(is0)
    c1 = jnp.cumsum(1 - is0)
    n0 = c0[-1]
    ceil0 = (n0 + QB - 1) // QB
    padpos = jnp.where(style == 0, c0 - 1, ceil0 * QB + c1 - 1)
    perm = jnp.zeros((nb * QB,), jnp.int32).at[padpos].set(
        jnp.arange(q_n, dtype=jnp.int32), mode="drop")
    qs = batch_query[perm]
    bstyle = (jnp.arange(nb, dtype=jnp.int32) >= ceil0).astype(jnp.int32)

    body = functools.partial(_body, n_k=k_n, nkb=nkb)
    grid_spec = pltpu.PrefetchScalarGridSpec(
        num_scalar_prefetch=1,
        grid=(nb, nkb),
        in_specs=[
            pl.BlockSpec((QB, d), lambda b, kb, bs: (b, 0)),
            pl.BlockSpec((1, KBLK, d), lambda b, kb, bs: (bs[b], kb, 0)),
        ],
        out_specs=(
            pl.BlockSpec((QB, TOPK), lambda b, kb, bs: (b, 0)),
            pl.BlockSpec((QB, TOPK), lambda b, kb, bs: (b, 0)),
        ),
        scratch_shapes=[
            pltpu.VMEM((NLVL, QB, LANES), jnp.float32),
            pltpu.VMEM((NLVL, QB, LANES), jnp.int32),
        ],
    )
    vals_p, idx_p = pl.pallas_call(
        body,
        grid_spec=grid_spec,
        out_shape=(
            jax.ShapeDtypeStruct((nb * QB, TOPK), jnp.float32),
            jax.ShapeDtypeStruct((nb * QB, TOPK), jnp.int32),
        ),
        compiler_params=pltpu.CompilerParams(
            dimension_semantics=("parallel", "arbitrary")),
        interpret=False,
    )(bstyle, qs, keys)

    return vals_p[padpos], idx_p[padpos]


# speedup vs baseline: 1.4792x; 1.0007x over previous
import functools

import jax
import jax.numpy as jnp
from jax.experimental import pallas as pl
from jax.experimental.pallas import tpu as pltpu

QB = 256
KBLK = 25600
LANES = 128
NLVL = 4
TOPK = 5


def _body(bs_ref, q_ref, k_ref, vals_ref, idx_ref, accv_ref, acci_ref,
          *, n_k, nkb):
    kb = pl.program_id(1)

    @pl.when(kb == 0)
    def _init():
        accv_ref[...] = jnp.full(accv_ref.shape, -jnp.inf, jnp.float32)
        acci_ref[...] = jnp.zeros(acci_ref.shape, jnp.int32)

    q = q_ref[...]
    qn = q / jnp.sqrt(jnp.sum(q * q, axis=1, keepdims=True))
    k = k_ref[0]
    kn = k / jnp.sqrt(jnp.sum(k * k, axis=1, keepdims=True))
    scores = jax.lax.dot_general(
        qn, kn, (((1,), (1,)), ((), ())),
        preferred_element_type=jnp.float32)
    iot = jax.lax.broadcasted_iota(jnp.int32, (QB, LANES), 1)
    base = kb * KBLK
    safe_c = max((n_k - (nkb - 1) * KBLK) // LANES, 0)

    for c in range(KBLK // LANES):
        v = scores[:, c * LANES:(c + 1) * LANES]
        vi = iot + (base + c * LANES)
        if c >= safe_c:
            v = jnp.where(vi < n_k, v, -jnp.inf)
        for j in range(NLVL):
            av = accv_ref[j]
            ai = acci_ref[j]
            gt = v > av
            accv_ref[j] = jnp.where(gt, v, av)
            acci_ref[j] = jnp.where(gt, vi, ai)
            if j < NLVL - 1:
                v = jnp.where(gt, av, v)
                vi = jnp.where(gt, ai, vi)

    @pl.when(kb == nkb - 1)
    def _extract():
        Vw = [accv_ref[j] for j in range(NLVL)]
        Iw = [acci_ref[j] for j in range(NLVL)]
        outv, outi = [], []
        for _r in range(TOPK):
            M, MI = Vw[0], Iw[0]
            for j in range(1, NLVL):
                better = (Vw[j] > M) | ((Vw[j] == M) & (Iw[j] < MI))
                M = jnp.where(better, Vw[j], M)
                MI = jnp.where(better, Iw[j], MI)
            m = jnp.max(M, axis=1, keepdims=True)
            mi = jnp.min(jnp.where(M == m, MI, jnp.int32(2**31 - 1)),
                         axis=1, keepdims=True)
            outv.append(m)
            outi.append(mi)
            for j in range(NLVL):
                hit = (Vw[j] == m) & (Iw[j] == mi)
                Vw[j] = jnp.where(hit, -jnp.inf, Vw[j])
        vals_ref[...] = jnp.concatenate(outv, axis=1)
        idx_ref[...] = jnp.concatenate(outi, axis=1)


def kernel(batch_inputs, batch_query, batch_style, keys, topk):
    del batch_inputs, topk
    q_n, d = batch_query.shape
    s_n, k_n, _ = keys.shape
    nb = q_n // QB + 1
    nkb = (k_n + KBLK - 1) // KBLK

    style = batch_style.astype(jnp.int32)
    is0 = (style == 0).astype(jnp.int32)
    c0 = jnp.cumsum(is0)
    c1 = jnp.cumsum(1 - is0)
    n0 = c0[-1]
    ceil0 = (n0 + QB - 1) // QB
    padpos = jnp.where(style == 0, c0 - 1, ceil0 * QB + c1 - 1)
    perm = jnp.zeros((nb * QB,), jnp.int32).at[padpos].set(
        jnp.arange(q_n, dtype=jnp.int32), mode="drop")
    qs = batch_query[perm]
    bstyle = (jnp.arange(nb, dtype=jnp.int32) >= ceil0).astype(jnp.int32)

    body = functools.partial(_body, n_k=k_n, nkb=nkb)
    grid_spec = pltpu.PrefetchScalarGridSpec(
        num_scalar_prefetch=1,
        grid=(nb, nkb),
        in_specs=[
            pl.BlockSpec((QB, d), lambda b, kb, bs: (b, 0)),
            pl.BlockSpec((1, KBLK, d), lambda b, kb, bs: (bs[b], kb, 0)),
        ],
        out_specs=(
            pl.BlockSpec((QB, TOPK), lambda b, kb, bs: (b, 0)),
            pl.BlockSpec((QB, TOPK), lambda b, kb, bs: (b, 0)),
        ),
        scratch_shapes=[
            pltpu.VMEM((NLVL, QB, LANES), jnp.float32),
            pltpu.VMEM((NLVL, QB, LANES), jnp.int32),
        ],
    )
    vals_p, idx_p = pl.pallas_call(
        body,
        grid_spec=grid_spec,
        out_shape=(
            jax.ShapeDtypeStruct((nb * QB, TOPK), jnp.float32),
            jax.ShapeDtypeStruct((nb * QB, TOPK), jnp.int32),
        ),
        compiler_params=pltpu.CompilerParams(
            dimension_semantics=("parallel", "arbitrary")),
        interpret=False,
    )(bstyle, qs, keys)

    return vals_p[padpos], idx_p[padpos]


# speedup vs baseline: 1.5057x; 1.0179x over previous
import functools

import jax
import jax.numpy as jnp
from jax.experimental import pallas as pl
from jax.experimental.pallas import tpu as pltpu

QB = 256
KBLK = 25088
LANES = 128
NLVL = 4
TOPK = 5


def _body(bs_ref, q_ref, k_ref, vals_ref, idx_ref, accv_ref, acci_ref,
          *, n_k, nkb):
    kb = pl.program_id(1)

    @pl.when(kb == 0)
    def _init():
        accv_ref[...] = jnp.full(accv_ref.shape, -jnp.inf, jnp.float32)
        acci_ref[...] = jnp.zeros(acci_ref.shape, jnp.int32)

    q = q_ref[...]
    qn = q / jnp.sqrt(jnp.sum(q * q, axis=1, keepdims=True))
    k = k_ref[0]
    kn = k / jnp.sqrt(jnp.sum(k * k, axis=1, keepdims=True))
    scores = jax.lax.dot_general(
        qn, kn, (((1,), (1,)), ((), ())),
        preferred_element_type=jnp.float32)
    iot = jax.lax.broadcasted_iota(jnp.int32, (QB, LANES), 1)
    base = kb * KBLK
    safe_c = max((n_k - (nkb - 1) * KBLK) // LANES, 0)

    for c in range(KBLK // LANES):
        v = scores[:, c * LANES:(c + 1) * LANES]
        vi = iot + (base + c * LANES)
        if c >= safe_c:
            v = jnp.where(vi < n_k, v, -jnp.inf)
        for j in range(NLVL):
            av = accv_ref[j]
            ai = acci_ref[j]
            gt = v > av
            accv_ref[j] = jnp.where(gt, v, av)
            acci_ref[j] = jnp.where(gt, vi, ai)
            if j < NLVL - 1:
                v = jnp.where(gt, av, v)
                vi = jnp.where(gt, ai, vi)

    @pl.when(kb == nkb - 1)
    def _extract():
        Vw = [accv_ref[j] for j in range(NLVL)]
        Iw = [acci_ref[j] for j in range(NLVL)]
        outv, outi = [], []
        for _r in range(TOPK):
            M, MI = Vw[0], Iw[0]
            for j in range(1, NLVL):
                better = (Vw[j] > M) | ((Vw[j] == M) & (Iw[j] < MI))
                M = jnp.where(better, Vw[j], M)
                MI = jnp.where(better, Iw[j], MI)
            m = jnp.max(M, axis=1, keepdims=True)
            mi = jnp.min(jnp.where(M == m, MI, jnp.int32(2**31 - 1)),
                         axis=1, keepdims=True)
            outv.append(m)
            outi.append(mi)
            for j in range(NLVL):
                hit = (Vw[j] == m) & (Iw[j] == mi)
                Vw[j] = jnp.where(hit, -jnp.inf, Vw[j])
        vals_ref[...] = jnp.concatenate(outv, axis=1)
        idx_ref[...] = jnp.concatenate(outi, axis=1)


def kernel(batch_inputs, batch_query, batch_style, keys, topk):
    del batch_inputs, topk
    q_n, d = batch_query.shape
    s_n, k_n, _ = keys.shape
    nb = q_n // QB + 1
    nkb = (k_n + KBLK - 1) // KBLK

    style = batch_style.astype(jnp.int32)
    is0 = (style == 0).astype(jnp.int32)
    c0 = jnp.cumsum(is0)
    c1 = jnp.cumsum(1 - is0)
    n0 = c0[-1]
    ceil0 = (n0 + QB - 1) // QB
    padpos = jnp.where(style == 0, c0 - 1, ceil0 * QB + c1 - 1)
    perm = jnp.zeros((nb * QB,), jnp.int32).at[padpos].set(
        jnp.arange(q_n, dtype=jnp.int32), mode="drop")
    qs = batch_query[perm]
    bstyle = (jnp.arange(nb, dtype=jnp.int32) >= ceil0).astype(jnp.int32)

    body = functools.partial(_body, n_k=k_n, nkb=nkb)
    grid_spec = pltpu.PrefetchScalarGridSpec(
        num_scalar_prefetch=1,
        grid=(nb, nkb),
        in_specs=[
            pl.BlockSpec((QB, d), lambda b, kb, bs: (b, 0)),
            pl.BlockSpec((1, KBLK, d), lambda b, kb, bs: (bs[b], kb, 0)),
        ],
        out_specs=(
            pl.BlockSpec((QB, TOPK), lambda b, kb, bs: (b, 0)),
            pl.BlockSpec((QB, TOPK), lambda b, kb, bs: (b, 0)),
        ),
        scratch_shapes=[
            pltpu.VMEM((NLVL, QB, LANES), jnp.float32),
            pltpu.VMEM((NLVL, QB, LANES), jnp.int32),
        ],
    )
    vals_p, idx_p = pl.pallas_call(
        body,
        grid_spec=grid_spec,
        out_shape=(
            jax.ShapeDtypeStruct((nb * QB, TOPK), jnp.float32),
            jax.ShapeDtypeStruct((nb * QB, TOPK), jnp.int32),
        ),
        compiler_params=pltpu.CompilerParams(
            dimension_semantics=("parallel", "arbitrary")),
        interpret=False,
    )(bstyle, qs, keys)

    return vals_p[padpos], idx_p[padpos]


# speedup vs baseline: 1.8298x; 1.2153x over previous
import functools

import jax
import jax.numpy as jnp
from jax.experimental import pallas as pl
from jax.experimental.pallas import tpu as pltpu

QB = 352
KBLK = 12544
LANES = 128
NLVL = 4
TOPK = 5


def _body(bs_ref, q_ref, k_ref, vals_ref, idx_ref, accv_ref, acci_ref,
          *, n_k, nkb):
    kb = pl.program_id(1)

    @pl.when(kb == 0)
    def _init():
        accv_ref[...] = jnp.full(accv_ref.shape, -jnp.inf, jnp.float32)
        acci_ref[...] = jnp.zeros(acci_ref.shape, jnp.int32)

    q = q_ref[...]
    qn = q / jnp.sqrt(jnp.sum(q * q, axis=1, keepdims=True))
    k = k_ref[0]
    kn = k / jnp.sqrt(jnp.sum(k * k, axis=1, keepdims=True))
    scores = jax.lax.dot_general(
        qn, kn, (((1,), (1,)), ((), ())),
        preferred_element_type=jnp.float32)
    iot = jax.lax.broadcasted_iota(jnp.int32, (QB, LANES), 1)
    base = kb * KBLK
    safe_c = max((n_k - (nkb - 1) * KBLK) // LANES, 0)

    for c in range(KBLK // LANES):
        v = scores[:, c * LANES:(c + 1) * LANES]
        vi = iot + (base + c * LANES)
        if c >= safe_c:
            v = jnp.where(vi < n_k, v, -jnp.inf)
        for j in range(NLVL):
            av = accv_ref[j]
            ai = acci_ref[j]
            gt = v > av
            accv_ref[j] = jnp.where(gt, v, av)
            acci_ref[j] = jnp.where(gt, vi, ai)
            if j < NLVL - 1:
                v = jnp.where(gt, av, v)
                vi = jnp.where(gt, ai, vi)

    @pl.when(kb == nkb - 1)
    def _extract():
        Vw = [accv_ref[j] for j in range(NLVL)]
        Iw = [acci_ref[j] for j in range(NLVL)]
        outv, outi = [], []
        for _r in range(TOPK):
            M, MI = Vw[0], Iw[0]
            for j in range(1, NLVL):
                better = (Vw[j] > M) | ((Vw[j] == M) & (Iw[j] < MI))
                M = jnp.where(better, Vw[j], M)
                MI = jnp.where(better, Iw[j], MI)
            m = jnp.max(M, axis=1, keepdims=True)
            mi = jnp.min(jnp.where(M == m, MI, jnp.int32(2**31 - 1)),
                         axis=1, keepdims=True)
            outv.append(m)
            outi.append(mi)
            for j in range(NLVL):
                hit = (Vw[j] == m) & (Iw[j] == mi)
                Vw[j] = jnp.where(hit, -jnp.inf, Vw[j])
        vals_ref[...] = jnp.concatenate(outv, axis=1)
        idx_ref[...] = jnp.concatenate(outi, axis=1)


def kernel(batch_inputs, batch_query, batch_style, keys, topk):
    del batch_inputs, topk
    q_n, d = batch_query.shape
    s_n, k_n, _ = keys.shape
    nb = q_n // QB + 1
    nkb = (k_n + KBLK - 1) // KBLK

    style = batch_style.astype(jnp.int32)
    is0 = (style == 0).astype(jnp.int32)
    c0 = jnp.cumsum(is0)
    c1 = jnp.cumsum(1 - is0)
    n0 = c0[-1]
    ceil0 = (n0 + QB - 1) // QB
    padpos = jnp.where(style == 0, c0 - 1, ceil0 * QB + c1 - 1)
    perm = jnp.zeros((nb * QB,), jnp.int32).at[padpos].set(
        jnp.arange(q_n, dtype=jnp.int32), mode="drop")
    qs = batch_query[perm]
    bstyle = (jnp.arange(nb, dtype=jnp.int32) >= ceil0).astype(jnp.int32)

    body = functools.partial(_body, n_k=k_n, nkb=nkb)
    grid_spec = pltpu.PrefetchScalarGridSpec(
        num_scalar_prefetch=1,
        grid=(nb, nkb),
        in_specs=[
            pl.BlockSpec((QB, d), lambda b, kb, bs: (b, 0)),
            pl.BlockSpec((1, KBLK, d), lambda b, kb, bs: (bs[b], kb, 0)),
        ],
        out_specs=(
            pl.BlockSpec((QB, TOPK), lambda b, kb, bs: (b, 0)),
            pl.BlockSpec((QB, TOPK), lambda b, kb, bs: (b, 0)),
        ),
        scratch_shapes=[
            pltpu.VMEM((NLVL, QB, LANES), jnp.float32),
            pltpu.VMEM((NLVL, QB, LANES), jnp.int32),
        ],
    )
    vals_p, idx_p = pl.pallas_call(
        body,
        grid_spec=grid_spec,
        out_shape=(
            jax.ShapeDtypeStruct((nb * QB, TOPK), jnp.float32),
            jax.ShapeDtypeStruct((nb * QB, TOPK), jnp.int32),
        ),
        compiler_params=pltpu.CompilerParams(
            dimension_semantics=("parallel", "arbitrary")),
        interpret=False,
    )(bstyle, qs, keys)

    return vals_p[padpos], idx_p[padpos]
